# Initial kernel scaffold; baseline (speedup 1.0000x reference)
import jax
import jax.numpy as jnp
from jax.experimental import pallas as pl


def kernel(x, edge_index, W1l, b1, W1r, W2l, b2, W2r):
    raise NotImplementedError("write your pallas kernel here")


# speedup vs baseline: 4.6013x; 4.6013x over previous
import functools

import jax
import jax.numpy as jnp
from jax import lax
from jax.experimental import pallas as pl
from jax.experimental.pallas import tpu as pltpu
from jax.experimental.pallas import tpu_sc as plsc

N = 10000
E = 320000
D_IN = 166
D_H = 128
D_OUT = 2

NP = 10240
KP = 256
DP = 16

NC = 2
NS = 16
NW = NC * NS

CH1 = 256
CH2 = 256
EP = 327680
EPW = EP // NW
NITER1 = EPW // CH1
NITER2 = EPW // CH2
STRIPE = NP // NS

BN = 512
GRID = NP // BN


def _tc_a_body(x_ref, wl_ref, wr_ref, b_ref, y1_ref, r1_ref):
    x = x_ref[...]
    y1_ref[...] = jnp.dot(x, wl_ref[...], preferred_element_type=jnp.float32)
    r1_ref[...] = (
        jnp.dot(x, wr_ref[...], preferred_element_type=jnp.float32) + b_ref[...]
    )


def _tc_a(xp, wl, wr, b):
    return pl.pallas_call(
        _tc_a_body,
        grid=(GRID,),
        in_specs=[
            pl.BlockSpec((BN, KP), lambda i: (i, 0)),
            pl.BlockSpec((KP, D_H), lambda i: (0, 0)),
            pl.BlockSpec((KP, D_H), lambda i: (0, 0)),
            pl.BlockSpec((1, D_H), lambda i: (0, 0)),
        ],
        out_specs=[
            pl.BlockSpec((BN, D_H), lambda i: (i, 0)),
            pl.BlockSpec((BN, D_H), lambda i: (i, 0)),
        ],
        out_shape=[
            jax.ShapeDtypeStruct((NP, D_H), jnp.float32),
            jax.ShapeDtypeStruct((NP, D_H), jnp.float32),
        ],
    )(xp, wl, wr, b)


def _sc_b_body(y1_hbm, src_hbm, dst_hbm, z128_hbm,
               agg_out,
               src_v, dst_v, rows_v, agg_sh, sem):
    c = lax.axis_index("c")
    s = lax.axis_index("s")
    wid = s * NC + c
    pltpu.sync_copy(z128_hbm, agg_sh.at[pl.ds(s * STRIPE, STRIPE)])
    plsc.subcore_barrier()

    def body(g, carry):
        base = wid * EPW + g * CH1
        pltpu.sync_copy(src_hbm.at[pl.ds(base, CH1)], src_v)
        pltpu.sync_copy(dst_hbm.at[pl.ds(base, CH1)], dst_v)
        pltpu.async_copy(y1_hbm.at[src_v], rows_v, sem).wait()
        pltpu.sync_copy(rows_v, agg_sh.at[dst_v], add=True)
        return carry

    lax.fori_loop(0, NITER1, body, 0)
    plsc.subcore_barrier()
    pltpu.sync_copy(agg_sh.at[pl.ds(s * STRIPE, STRIPE)],
                    agg_out.at[c, pl.ds(s * STRIPE, STRIPE)])


_sc_b = functools.partial(
    pl.kernel,
    mesh=plsc.VectorSubcoreMesh(core_axis_name="c", subcore_axis_name="s"),
    out_type=jax.ShapeDtypeStruct((NC, NP, D_H), jnp.float32),
    scratch_types=[
        pltpu.VMEM((CH1,), jnp.int32),
        pltpu.VMEM((CH1,), jnp.int32),
        pltpu.VMEM((CH1, D_H), jnp.float32),
        pltpu.VMEM_SHARED((NP, D_H), jnp.float32),
        pltpu.SemaphoreType.DMA,
    ],
)(_sc_b_body)


def _sc_cnt_body(dst_hbm, z16_hbm, ones_hbm,
                 cnt_out,
                 dst_v, ones_v, cnt_sh, sem):
    c = lax.axis_index("c")
    s = lax.axis_index("s")
    wid = s * NC + c
    pltpu.sync_copy(z16_hbm, cnt_sh.at[pl.ds(s * STRIPE, STRIPE)])
    pltpu.sync_copy(ones_hbm, ones_v)
    plsc.subcore_barrier()

    def body(g, carry):
        base = wid * EPW + g * CH1
        pltpu.sync_copy(dst_hbm.at[pl.ds(base, CH1)], dst_v)
        pltpu.sync_copy(ones_v, cnt_sh.at[dst_v], add=True)
        return carry

    lax.fori_loop(0, NITER1, body, 0)
    plsc.subcore_barrier()
    pltpu.sync_copy(cnt_sh.at[pl.ds(s * STRIPE, STRIPE)],
                    cnt_out.at[c, pl.ds(s * STRIPE, STRIPE)])


_sc_cnt = functools.partial(
    pl.kernel,
    mesh=plsc.VectorSubcoreMesh(core_axis_name="c", subcore_axis_name="s"),
    out_type=jax.ShapeDtypeStruct((NC, NP, DP), jnp.float32),
    compiler_params=pltpu.CompilerParams(use_tc_tiling_on_sc=False),
    scratch_types=[
        pltpu.VMEM((CH1,), jnp.int32),
        pltpu.VMEM((CH1, DP), jnp.float32),
        pltpu.VMEM_SHARED((NP, DP), jnp.float32),
        pltpu.SemaphoreType.DMA,
    ],
)(_sc_cnt_body)


def _tc_c_body(agg_ref, cnt_ref, r1_ref, wl_ref, wr_ref, b_ref,
               y2_ref, r2_ref):
    cnt = cnt_ref[0, :, 0:1] + cnt_ref[1, :, 0:1]
    mean = (agg_ref[0] + agg_ref[1]) / jnp.maximum(cnt, 1.0)
    h = jnp.maximum(mean + r1_ref[...], 0.0)
    y2_ref[...] = jnp.dot(h, wl_ref[...], preferred_element_type=jnp.float32)
    r2_ref[...] = (
        jnp.dot(h, wr_ref[...], preferred_element_type=jnp.float32) + b_ref[...]
    )


def _tc_c(aggp, cntp, r1, wl, wr, b):
    return pl.pallas_call(
        _tc_c_body,
        grid=(GRID,),
        in_specs=[
            pl.BlockSpec((NC, BN, D_H), lambda i: (0, i, 0)),
            pl.BlockSpec((NC, BN, DP), lambda i: (0, i, 0)),
            pl.BlockSpec((BN, D_H), lambda i: (i, 0)),
            pl.BlockSpec((D_H, DP), lambda i: (0, 0)),
            pl.BlockSpec((D_H, DP), lambda i: (0, 0)),
            pl.BlockSpec((1, DP), lambda i: (0, 0)),
        ],
        out_specs=[
            pl.BlockSpec((BN, DP), lambda i: (i, 0)),
            pl.BlockSpec((BN, DP), lambda i: (i, 0)),
        ],
        out_shape=[
            jax.ShapeDtypeStruct((NP, DP), jnp.float32),
            jax.ShapeDtypeStruct((NP, DP), jnp.float32),
        ],
    )(aggp, cntp, r1, wl, wr, b)


def _sc_d_body(y2_hbm, src_hbm, dst_hbm, z16_hbm,
               acc_out,
               src_v, dst_v, rows_v, acc_sh, sem):
    c = lax.axis_index("c")
    s = lax.axis_index("s")
    wid = s * NC + c
    pltpu.sync_copy(z16_hbm, acc_sh.at[pl.ds(s * STRIPE, STRIPE)])
    plsc.subcore_barrier()

    def body(g, carry):
        base = wid * EPW + g * CH2
        pltpu.sync_copy(src_hbm.at[pl.ds(base, CH2)], src_v)
        pltpu.sync_copy(dst_hbm.at[pl.ds(base, CH2)], dst_v)
        pltpu.async_copy(y2_hbm.at[src_v], rows_v, sem).wait()
        pltpu.sync_copy(rows_v, acc_sh.at[dst_v], add=True)
        return carry

    lax.fori_loop(0, NITER2, body, 0)
    plsc.subcore_barrier()
    pltpu.sync_copy(acc_sh.at[pl.ds(s * STRIPE, STRIPE)],
                    acc_out.at[c, pl.ds(s * STRIPE, STRIPE)])


_sc_d = functools.partial(
    pl.kernel,
    mesh=plsc.VectorSubcoreMesh(core_axis_name="c", subcore_axis_name="s"),
    out_type=jax.ShapeDtypeStruct((NC, NP, DP), jnp.float32),
    compiler_params=pltpu.CompilerParams(use_tc_tiling_on_sc=False),
    scratch_types=[
        pltpu.VMEM((CH2,), jnp.int32),
        pltpu.VMEM((CH2,), jnp.int32),
        pltpu.VMEM((CH2, DP), jnp.float32),
        pltpu.VMEM_SHARED((NP, DP), jnp.float32),
        pltpu.SemaphoreType.DMA,
    ],
)(_sc_d_body)


def _tc_e_body(acc_ref, cnt_ref, r2_ref, o_ref):
    cnt = cnt_ref[0, :, 0:1] + cnt_ref[1, :, 0:1]
    z = (acc_ref[0] + acc_ref[1]) / jnp.maximum(cnt, 1.0) + r2_ref[...]
    z0 = z[:, 0:1]
    z1 = z[:, 1:2]
    m = jnp.maximum(z0, z1)
    lse = m + jnp.log(jnp.exp(z0 - m) + jnp.exp(z1 - m))
    o_ref[...] = z - lse


def _tc_e(accp, cntp, r2):
    return pl.pallas_call(
        _tc_e_body,
        grid=(GRID,),
        in_specs=[
            pl.BlockSpec((NC, BN, DP), lambda i: (0, i, 0)),
            pl.BlockSpec((NC, BN, DP), lambda i: (0, i, 0)),
            pl.BlockSpec((BN, DP), lambda i: (i, 0)),
        ],
        out_specs=pl.BlockSpec((BN, DP), lambda i: (i, 0)),
        out_shape=jax.ShapeDtypeStruct((NP, DP), jnp.float32),
    )(accp, cntp, r2)


def kernel(x, edge_index, W1l, b1, W1r, W2l, b2, W2r):
    xp = jnp.pad(x, ((0, NP - N), (0, KP - D_IN)))
    w1l = jnp.pad(W1l, ((0, KP - D_IN), (0, 0)))
    w1r = jnp.pad(W1r, ((0, KP - D_IN), (0, 0)))
    b1p = b1[None, :]
    w2l = jnp.pad(W2l, ((0, 0), (0, DP - D_OUT)))
    w2r = jnp.pad(W2r, ((0, 0), (0, DP - D_OUT)))
    b2p = jnp.pad(b2, (0, DP - D_OUT))[None, :]

    srcp = jnp.full((EP,), N, jnp.int32).at[:E].set(edge_index[0])
    dstp = jnp.full((EP,), N, jnp.int32).at[:E].set(edge_index[1])

    z128 = jnp.zeros((STRIPE, D_H), jnp.float32)
    z16 = jnp.zeros((STRIPE, DP), jnp.float32)
    ones = jnp.ones((CH1, DP), jnp.float32)

    y1, r1 = _tc_a(xp, w1l, w1r, b1p)
    cntp = _sc_cnt(dstp, z16, ones)
    aggp = _sc_b(y1, srcp, dstp, z128)
    y2, r2 = _tc_c(aggp, cntp, r1, w2l, w2r, b2p)
    accp = _sc_d(y2, srcp, dstp, z16)
    o = _tc_e(accp, cntp, r2)
    return o[:N, :D_OUT]


# speedup vs baseline: 5.1372x; 1.1165x over previous
import functools

import jax
import jax.numpy as jnp
from jax import lax
from jax.experimental import pallas as pl
from jax.experimental.pallas import tpu as pltpu
from jax.experimental.pallas import tpu_sc as plsc

N = 10000
E = 320000
D_IN = 166
D_H = 128
D_OUT = 2

NP = 10240
KP = 256
DP = 16

NC = 2
NS = 16
NW = NC * NS

CH = 128
EP = 327680
EPW = EP // NW
NCHW = EPW // CH
SB1 = 2
SBCH = NCHW // SB1
CH1 = 256
NITER1 = EPW // CH1
STRIPE = NP // NS

BN = 512
GRID = NP // BN


def _tc_a_body(x_ref, wl_ref, wr_ref, b_ref, y1_ref, r1_ref):
    x = x_ref[...]
    y1_ref[...] = jnp.dot(x, wl_ref[...], preferred_element_type=jnp.float32)
    r1_ref[...] = (
        jnp.dot(x, wr_ref[...], preferred_element_type=jnp.float32) + b_ref[...]
    )


def _tc_a(xp, wl, wr, b):
    return pl.pallas_call(
        _tc_a_body,
        grid=(GRID,),
        in_specs=[
            pl.BlockSpec((BN, KP), lambda i: (i, 0)),
            pl.BlockSpec((KP, D_H), lambda i: (0, 0)),
            pl.BlockSpec((KP, D_H), lambda i: (0, 0)),
            pl.BlockSpec((1, D_H), lambda i: (0, 0)),
        ],
        out_specs=[
            pl.BlockSpec((BN, D_H), lambda i: (i, 0)),
            pl.BlockSpec((BN, D_H), lambda i: (i, 0)),
        ],
        out_shape=[
            jax.ShapeDtypeStruct((NP, D_H), jnp.float32),
            jax.ShapeDtypeStruct((NP, D_H), jnp.float32),
        ],
    )(xp, wl, wr, b)


def _sc_b_body(y1_hbm, src_hbm, dst_hbm, z128_hbm,
               agg_out,
               srcm, dstm, rows0, rows1, agg_sh, sem0, sem1):
    c = lax.axis_index("c")
    s = lax.axis_index("s")
    wid = s * NC + c
    pltpu.sync_copy(z128_hbm, agg_sh.at[pl.ds(s * STRIPE, STRIPE)])
    plsc.subcore_barrier()

    for sb in range(SB1):
        rb = wid * NCHW + sb * SBCH
        pltpu.sync_copy(src_hbm.at[pl.ds(rb, SBCH)], srcm)
        pltpu.sync_copy(dst_hbm.at[pl.ds(rb, SBCH)], dstm)
        pltpu.async_copy(y1_hbm.at[srcm.at[0]], rows0, sem0)

        def body(k, carry):
            pltpu.async_copy(y1_hbm.at[srcm.at[2 * k + 1]], rows1, sem1)
            pltpu.make_async_copy(y1_hbm.at[srcm.at[2 * k]], rows0, sem0).wait()
            pltpu.sync_copy(rows0, agg_sh.at[dstm.at[2 * k]], add=True)

            @pl.when(k < SBCH // 2 - 1)
            def _():
                pltpu.async_copy(y1_hbm.at[srcm.at[2 * k + 2]], rows0, sem0)

            pltpu.make_async_copy(y1_hbm.at[srcm.at[2 * k + 1]], rows1,
                                  sem1).wait()
            pltpu.sync_copy(rows1, agg_sh.at[dstm.at[2 * k + 1]], add=True)
            return carry

        lax.fori_loop(0, SBCH // 2, body, 0)

    plsc.subcore_barrier()
    pltpu.sync_copy(agg_sh.at[pl.ds(s * STRIPE, STRIPE)],
                    agg_out.at[c, pl.ds(s * STRIPE, STRIPE)])


_sc_b = functools.partial(
    pl.kernel,
    mesh=plsc.VectorSubcoreMesh(core_axis_name="c", subcore_axis_name="s"),
    out_type=jax.ShapeDtypeStruct((NC, NP, D_H), jnp.float32),
    scratch_types=[
        pltpu.VMEM((SBCH, CH), jnp.int32),
        pltpu.VMEM((SBCH, CH), jnp.int32),
        pltpu.VMEM((CH, D_H), jnp.float32),
        pltpu.VMEM((CH, D_H), jnp.float32),
        pltpu.VMEM_SHARED((NP, D_H), jnp.float32),
        pltpu.SemaphoreType.DMA,
        pltpu.SemaphoreType.DMA,
    ],
)(_sc_b_body)


def _sc_cnt_body(dst_hbm, z16_hbm, ones_hbm,
                 cnt_out,
                 dst_v, ones_v, cnt_sh, sem):
    c = lax.axis_index("c")
    s = lax.axis_index("s")
    wid = s * NC + c
    pltpu.sync_copy(z16_hbm, cnt_sh.at[pl.ds(s * STRIPE, STRIPE)])
    pltpu.sync_copy(ones_hbm, ones_v)
    plsc.subcore_barrier()

    def body(g, carry):
        base = wid * EPW + g * CH1
        pltpu.sync_copy(dst_hbm.at[pl.ds(base, CH1)], dst_v)
        pltpu.sync_copy(ones_v, cnt_sh.at[dst_v], add=True)
        return carry

    lax.fori_loop(0, NITER1, body, 0)
    plsc.subcore_barrier()
    pltpu.sync_copy(cnt_sh.at[pl.ds(s * STRIPE, STRIPE)],
                    cnt_out.at[c, pl.ds(s * STRIPE, STRIPE)])


_sc_cnt = functools.partial(
    pl.kernel,
    mesh=plsc.VectorSubcoreMesh(core_axis_name="c", subcore_axis_name="s"),
    out_type=jax.ShapeDtypeStruct((NC, NP, DP), jnp.float32),
    compiler_params=pltpu.CompilerParams(use_tc_tiling_on_sc=False),
    scratch_types=[
        pltpu.VMEM((CH1,), jnp.int32),
        pltpu.VMEM((CH1, DP), jnp.float32),
        pltpu.VMEM_SHARED((NP, DP), jnp.float32),
        pltpu.SemaphoreType.DMA,
    ],
)(_sc_cnt_body)


def _tc_c_body(agg_ref, cnt_ref, r1_ref, wl_ref, wr_ref, b_ref,
               y2_ref, r2_ref):
    cnt = cnt_ref[0, :, 0:1] + cnt_ref[1, :, 0:1]
    mean = (agg_ref[0] + agg_ref[1]) / jnp.maximum(cnt, 1.0)
    h = jnp.maximum(mean + r1_ref[...], 0.0)
    y2_ref[...] = jnp.dot(h, wl_ref[...], preferred_element_type=jnp.float32)
    r2_ref[...] = (
        jnp.dot(h, wr_ref[...], preferred_element_type=jnp.float32) + b_ref[...]
    )


def _tc_c(aggp, cntp, r1, wl, wr, b):
    return pl.pallas_call(
        _tc_c_body,
        grid=(GRID,),
        in_specs=[
            pl.BlockSpec((NC, BN, D_H), lambda i: (0, i, 0)),
            pl.BlockSpec((NC, BN, DP), lambda i: (0, i, 0)),
            pl.BlockSpec((BN, D_H), lambda i: (i, 0)),
            pl.BlockSpec((D_H, DP), lambda i: (0, 0)),
            pl.BlockSpec((D_H, DP), lambda i: (0, 0)),
            pl.BlockSpec((1, DP), lambda i: (0, 0)),
        ],
        out_specs=[
            pl.BlockSpec((BN, DP), lambda i: (i, 0)),
            pl.BlockSpec((BN, DP), lambda i: (i, 0)),
        ],
        out_shape=[
            jax.ShapeDtypeStruct((NP, DP), jnp.float32),
            jax.ShapeDtypeStruct((NP, DP), jnp.float32),
        ],
    )(aggp, cntp, r1, wl, wr, b)


def _sc_d_body(y2_hbm, src_hbm, dst_hbm, z16_hbm,
               acc_out,
               srcm, dstm, rows0, rows1, acc_sh, sem0, sem1):
    c = lax.axis_index("c")
    s = lax.axis_index("s")
    wid = s * NC + c
    pltpu.sync_copy(z16_hbm, acc_sh.at[pl.ds(s * STRIPE, STRIPE)])
    plsc.subcore_barrier()

    rb = wid * NCHW
    pltpu.sync_copy(src_hbm.at[pl.ds(rb, NCHW)], srcm)
    pltpu.sync_copy(dst_hbm.at[pl.ds(rb, NCHW)], dstm)
    pltpu.async_copy(y2_hbm.at[srcm.at[0]], rows0, sem0)

    def body(k, carry):
        pltpu.async_copy(y2_hbm.at[srcm.at[2 * k + 1]], rows1, sem1)
        pltpu.make_async_copy(y2_hbm.at[srcm.at[2 * k]], rows0, sem0).wait()
        pltpu.sync_copy(rows0, acc_sh.at[dstm.at[2 * k]], add=True)

        @pl.when(k < NCHW // 2 - 1)
        def _():
            pltpu.async_copy(y2_hbm.at[srcm.at[2 * k + 2]], rows0, sem0)

        pltpu.make_async_copy(y2_hbm.at[srcm.at[2 * k + 1]], rows1,
                              sem1).wait()
        pltpu.sync_copy(rows1, acc_sh.at[dstm.at[2 * k + 1]], add=True)
        return carry

    lax.fori_loop(0, NCHW // 2, body, 0)
    plsc.subcore_barrier()
    pltpu.sync_copy(acc_sh.at[pl.ds(s * STRIPE, STRIPE)],
                    acc_out.at[c, pl.ds(s * STRIPE, STRIPE)])


_sc_d = functools.partial(
    pl.kernel,
    mesh=plsc.VectorSubcoreMesh(core_axis_name="c", subcore_axis_name="s"),
    out_type=jax.ShapeDtypeStruct((NC, NP, DP), jnp.float32),
    compiler_params=pltpu.CompilerParams(use_tc_tiling_on_sc=False),
    scratch_types=[
        pltpu.VMEM((NCHW, CH), jnp.int32),
        pltpu.VMEM((NCHW, CH), jnp.int32),
        pltpu.VMEM((CH, DP), jnp.float32),
        pltpu.VMEM((CH, DP), jnp.float32),
        pltpu.VMEM_SHARED((NP, DP), jnp.float32),
        pltpu.SemaphoreType.DMA,
        pltpu.SemaphoreType.DMA,
    ],
)(_sc_d_body)


def _tc_e_body(acc_ref, cnt_ref, r2_ref, o_ref):
    cnt = cnt_ref[0, :, 0:1] + cnt_ref[1, :, 0:1]
    z = (acc_ref[0] + acc_ref[1]) / jnp.maximum(cnt, 1.0) + r2_ref[...]
    z0 = z[:, 0:1]
    z1 = z[:, 1:2]
    m = jnp.maximum(z0, z1)
    lse = m + jnp.log(jnp.exp(z0 - m) + jnp.exp(z1 - m))
    o_ref[...] = z - lse


def _tc_e(accp, cntp, r2):
    return pl.pallas_call(
        _tc_e_body,
        grid=(GRID,),
        in_specs=[
            pl.BlockSpec((NC, BN, DP), lambda i: (0, i, 0)),
            pl.BlockSpec((NC, BN, DP), lambda i: (0, i, 0)),
            pl.BlockSpec((BN, DP), lambda i: (i, 0)),
        ],
        out_specs=pl.BlockSpec((BN, DP), lambda i: (i, 0)),
        out_shape=jax.ShapeDtypeStruct((NP, DP), jnp.float32),
    )(accp, cntp, r2)


def kernel(x, edge_index, W1l, b1, W1r, W2l, b2, W2r):
    xp = jnp.pad(x, ((0, NP - N), (0, KP - D_IN)))
    w1l = jnp.pad(W1l, ((0, KP - D_IN), (0, 0)))
    w1r = jnp.pad(W1r, ((0, KP - D_IN), (0, 0)))
    b1p = b1[None, :]
    w2l = jnp.pad(W2l, ((0, 0), (0, DP - D_OUT)))
    w2r = jnp.pad(W2r, ((0, 0), (0, DP - D_OUT)))
    b2p = jnp.pad(b2, (0, DP - D_OUT))[None, :]

    srcp = jnp.full((EP,), N, jnp.int32).at[:E].set(edge_index[0])
    dstp = jnp.full((EP,), N, jnp.int32).at[:E].set(edge_index[1])
    src2 = srcp.reshape(EP // CH, CH)
    dst2 = dstp.reshape(EP // CH, CH)

    z128 = jnp.zeros((STRIPE, D_H), jnp.float32)
    z16 = jnp.zeros((STRIPE, DP), jnp.float32)
    ones = jnp.ones((CH1, DP), jnp.float32)

    y1, r1 = _tc_a(xp, w1l, w1r, b1p)
    cntp = _sc_cnt(dstp, z16, ones)
    aggp = _sc_b(y1, src2, dst2, z128)
    y2, r2 = _tc_c(aggp, cntp, r1, w2l, w2r, b2p)
    accp = _sc_d(y2, src2, dst2, z16)
    o = _tc_e(accp, cntp, r2)
    return o[:N, :D_OUT]


# speedup vs baseline: 13.5933x; 2.6461x over previous
import functools

import jax
import jax.numpy as jnp
from jax import lax
from jax.experimental import pallas as pl
from jax.experimental.pallas import tpu as pltpu
from jax.experimental.pallas import tpu_sc as plsc

N = 10000
E = 320000
D_IN = 166
D_H = 128
D_OUT = 2

NP = 10240
KP = 256
DP = 16

NC = 2
NS = 16
NW = NC * NS

CH = 128
EP = 327680
EPW = EP // NW
NCHW = EPW // CH
SB1 = 2
SBCH = NCHW // SB1
STRIPE = NP // NS

BN = 1024
GRID = NP // BN


def _tc_a_body(x_ref, wl_ref, wr_ref, b_ref, y1_ref, r1_ref):
    x = x_ref[...]
    y1_ref[...] = jnp.dot(x, wl_ref[...], preferred_element_type=jnp.float32)
    r1_ref[...] = (
        jnp.dot(x, wr_ref[...], preferred_element_type=jnp.float32) + b_ref[...]
    )


def _tc_a(xp, wl, wr, b):
    return pl.pallas_call(
        _tc_a_body,
        grid=(GRID,),
        in_specs=[
            pl.BlockSpec((BN, KP), lambda i: (i, 0)),
            pl.BlockSpec((KP, D_H), lambda i: (0, 0)),
            pl.BlockSpec((KP, D_H), lambda i: (0, 0)),
            pl.BlockSpec((1, D_H), lambda i: (0, 0)),
        ],
        out_specs=[
            pl.BlockSpec((BN, D_H), lambda i: (i, 0)),
            pl.BlockSpec((BN, D_H), lambda i: (i, 0)),
        ],
        out_shape=[
            jax.ShapeDtypeStruct((NP, D_H), jnp.float32),
            jax.ShapeDtypeStruct((NP, D_H), jnp.float32),
        ],
    )(xp, wl, wr, b)


def _sc_b_body(y1_hbm, src_hbm, dst_hbm, z128_hbm,
               agg_out,
               srcm, dstm, rows0, rows1, agg_sh, sem0, sem1):
    c = lax.axis_index("c")
    s = lax.axis_index("s")
    wid = s * NC + c
    pltpu.sync_copy(z128_hbm, agg_sh.at[pl.ds(s * STRIPE, STRIPE)])
    plsc.subcore_barrier()

    for sb in range(SB1):
        rb = wid * NCHW + sb * SBCH
        pltpu.sync_copy(src_hbm.at[pl.ds(rb, SBCH)], srcm)
        pltpu.sync_copy(dst_hbm.at[pl.ds(rb, SBCH)], dstm)
        pltpu.async_copy(y1_hbm.at[srcm.at[0]], rows0, sem0)

        def body(k, carry):
            pltpu.async_copy(y1_hbm.at[srcm.at[2 * k + 1]], rows1, sem1)
            pltpu.make_async_copy(y1_hbm.at[srcm.at[2 * k]], rows0, sem0).wait()
            pltpu.sync_copy(rows0, agg_sh.at[dstm.at[2 * k]], add=True)

            @pl.when(k < SBCH // 2 - 1)
            def _():
                pltpu.async_copy(y1_hbm.at[srcm.at[2 * k + 2]], rows0, sem0)

            pltpu.make_async_copy(y1_hbm.at[srcm.at[2 * k + 1]], rows1,
                                  sem1).wait()
            pltpu.sync_copy(rows1, agg_sh.at[dstm.at[2 * k + 1]], add=True)
            return carry

        lax.fori_loop(0, SBCH // 2, body, 0)

    plsc.subcore_barrier()
    pltpu.sync_copy(agg_sh.at[pl.ds(s * STRIPE, STRIPE)],
                    agg_out.at[c, pl.ds(s * STRIPE, STRIPE)])


_sc_b = functools.partial(
    pl.kernel,
    mesh=plsc.VectorSubcoreMesh(core_axis_name="c", subcore_axis_name="s"),
    out_type=jax.ShapeDtypeStruct((NC, NP, D_H), jnp.float32),
    scratch_types=[
        pltpu.VMEM((SBCH, CH), jnp.int32),
        pltpu.VMEM((SBCH, CH), jnp.int32),
        pltpu.VMEM((CH, D_H), jnp.float32),
        pltpu.VMEM((CH, D_H), jnp.float32),
        pltpu.VMEM_SHARED((NP, D_H), jnp.float32),
        pltpu.SemaphoreType.DMA,
        pltpu.SemaphoreType.DMA,
    ],
)(_sc_b_body)


def _sc_cnt_body(dst_hbm,
                 cnt_out,
                 dstm, ones_v, cnt_sh, sem):
    c = lax.axis_index("c")
    s = lax.axis_index("s")
    wid = s * NC + c

    def fill0(i, carry):
        ones_v[i] = jnp.zeros((DP,), jnp.float32)
        return carry

    lax.fori_loop(0, CH, fill0, 0)
    for j in range(STRIPE // CH):
        pltpu.sync_copy(ones_v,
                        cnt_sh.at[pl.ds(s * STRIPE + j * CH, CH)])

    def fill1(i, carry):
        ones_v[i] = jnp.ones((DP,), jnp.float32)
        return carry

    lax.fori_loop(0, CH, fill1, 0)
    pltpu.sync_copy(dst_hbm.at[pl.ds(wid * NCHW, NCHW)], dstm)
    plsc.subcore_barrier()

    def body(g, carry):
        pltpu.async_copy(ones_v, cnt_sh.at[dstm.at[g]], sem, add=True)
        return carry

    lax.fori_loop(0, NCHW, body, 0)

    def drain(g, carry):
        pltpu.make_async_copy(ones_v, cnt_sh.at[dstm.at[0]], sem).wait()
        return carry

    lax.fori_loop(0, NCHW, drain, 0)
    plsc.subcore_barrier()
    pltpu.sync_copy(cnt_sh.at[pl.ds(s * STRIPE, STRIPE)],
                    cnt_out.at[c, pl.ds(s * STRIPE, STRIPE)])


_sc_cnt = functools.partial(
    pl.kernel,
    mesh=plsc.VectorSubcoreMesh(core_axis_name="c", subcore_axis_name="s"),
    out_type=jax.ShapeDtypeStruct((NC, NP, DP), jnp.float32),
    compiler_params=pltpu.CompilerParams(use_tc_tiling_on_sc=False),
    scratch_types=[
        pltpu.VMEM((NCHW, CH), jnp.int32),
        pltpu.VMEM((CH, DP), jnp.float32),
        pltpu.VMEM_SHARED((NP, DP), jnp.float32),
        pltpu.SemaphoreType.DMA,
    ],
)(_sc_cnt_body)


def _tc_c_body(agg_ref, cnt_ref, r1_ref, wl_ref, wr_ref, b_ref,
               y2_ref, r2_ref):
    cnt = cnt_ref[0, :, 0:1] + cnt_ref[1, :, 0:1]
    mean = (agg_ref[0] + agg_ref[1]) / jnp.maximum(cnt, 1.0)
    h = jnp.maximum(mean + r1_ref[...], 0.0)
    y2_ref[...] = jnp.dot(h, wl_ref[...], preferred_element_type=jnp.float32)
    r2_ref[...] = (
        jnp.dot(h, wr_ref[...], preferred_element_type=jnp.float32) + b_ref[...]
    )


def _tc_c(aggp, cntp, r1, wl, wr, b):
    return pl.pallas_call(
        _tc_c_body,
        grid=(GRID,),
        in_specs=[
            pl.BlockSpec((NC, BN, D_H), lambda i: (0, i, 0)),
            pl.BlockSpec((NC, BN, DP), lambda i: (0, i, 0)),
            pl.BlockSpec((BN, D_H), lambda i: (i, 0)),
            pl.BlockSpec((D_H, DP), lambda i: (0, 0)),
            pl.BlockSpec((D_H, DP), lambda i: (0, 0)),
            pl.BlockSpec((1, DP), lambda i: (0, 0)),
        ],
        out_specs=[
            pl.BlockSpec((BN, DP), lambda i: (i, 0)),
            pl.BlockSpec((BN, DP), lambda i: (i, 0)),
        ],
        out_shape=[
            jax.ShapeDtypeStruct((NP, DP), jnp.float32),
            jax.ShapeDtypeStruct((NP, DP), jnp.float32),
        ],
    )(aggp, cntp, r1, wl, wr, b)


CHD = 512
NCHD = EPW // CHD


def _sc_d_body(y2_hbm, src_hbm, dst_hbm,
               acc_out,
               srcm, dstm, rows0, rows1, acc_sh, sem0, sem1):
    c = lax.axis_index("c")
    s = lax.axis_index("s")
    wid = s * NC + c

    def fill0(i, carry):
        rows0[i] = jnp.zeros((DP,), jnp.float32)
        return carry

    lax.fori_loop(0, CHD, fill0, 0)
    for j in range(STRIPE // CHD):
        pltpu.sync_copy(rows0.at[pl.ds(0, CHD)],
                        acc_sh.at[pl.ds(s * STRIPE + j * CHD, CHD)])
    pltpu.sync_copy(rows0.at[pl.ds(0, STRIPE - (STRIPE // CHD) * CHD)],
                    acc_sh.at[pl.ds(s * STRIPE + (STRIPE // CHD) * CHD,
                                    STRIPE - (STRIPE // CHD) * CHD)])
    plsc.subcore_barrier()

    rb = wid * NCHD
    pltpu.sync_copy(src_hbm.at[pl.ds(rb, NCHD)], srcm)
    pltpu.sync_copy(dst_hbm.at[pl.ds(rb, NCHD)], dstm)
    pltpu.async_copy(y2_hbm.at[srcm.at[0]], rows0, sem0)

    def body(k, carry):
        pltpu.async_copy(y2_hbm.at[srcm.at[2 * k + 1]], rows1, sem1)
        pltpu.make_async_copy(y2_hbm.at[srcm.at[2 * k]], rows0,
                              sem0).wait()
        pltpu.sync_copy(rows0, acc_sh.at[dstm.at[2 * k]], add=True)

        @pl.when(k < NCHD // 2 - 1)
        def _():
            pltpu.async_copy(y2_hbm.at[srcm.at[2 * k + 2]], rows0, sem0)

        pltpu.make_async_copy(y2_hbm.at[srcm.at[2 * k + 1]], rows1,
                              sem1).wait()
        pltpu.sync_copy(rows1, acc_sh.at[dstm.at[2 * k + 1]], add=True)
        return carry

    lax.fori_loop(0, NCHD // 2, body, 0)
    plsc.subcore_barrier()
    pltpu.sync_copy(acc_sh.at[pl.ds(s * STRIPE, STRIPE)],
                    acc_out.at[c, pl.ds(s * STRIPE, STRIPE)])


_sc_d = functools.partial(
    pl.kernel,
    mesh=plsc.VectorSubcoreMesh(core_axis_name="c", subcore_axis_name="s"),
    out_type=jax.ShapeDtypeStruct((NC, NP, DP), jnp.float32),
    compiler_params=pltpu.CompilerParams(use_tc_tiling_on_sc=False),
    scratch_types=[
        pltpu.VMEM((NCHD, CHD), jnp.int32),
        pltpu.VMEM((NCHD, CHD), jnp.int32),
        pltpu.VMEM((CHD, DP), jnp.float32),
        pltpu.VMEM((CHD, DP), jnp.float32),
        pltpu.VMEM_SHARED((NP, DP), jnp.float32),
        pltpu.SemaphoreType.DMA,
        pltpu.SemaphoreType.DMA,
    ],
)(_sc_d_body)


def _tc_e_body(acc_ref, cnt_ref, r2_ref, o_ref):
    cnt = cnt_ref[0, :, 0:1] + cnt_ref[1, :, 0:1]
    z = (acc_ref[0] + acc_ref[1]) / jnp.maximum(cnt, 1.0) + r2_ref[...]
    z0 = z[:, 0:1]
    z1 = z[:, 1:2]
    m = jnp.maximum(z0, z1)
    lse = m + jnp.log(jnp.exp(z0 - m) + jnp.exp(z1 - m))
    o_ref[...] = z - lse


def _tc_e(accp, cntp, r2):
    return pl.pallas_call(
        _tc_e_body,
        grid=(GRID,),
        in_specs=[
            pl.BlockSpec((NC, BN, DP), lambda i: (0, i, 0)),
            pl.BlockSpec((NC, BN, DP), lambda i: (0, i, 0)),
            pl.BlockSpec((BN, DP), lambda i: (i, 0)),
        ],
        out_specs=pl.BlockSpec((BN, DP), lambda i: (i, 0)),
        out_shape=jax.ShapeDtypeStruct((NP, DP), jnp.float32),
    )(accp, cntp, r2)


def kernel(x, edge_index, W1l, b1, W1r, W2l, b2, W2r):
    xp = jnp.pad(x, ((0, NP - N), (0, KP - D_IN)))
    w1l = jnp.pad(W1l, ((0, KP - D_IN), (0, 0)))
    w1r = jnp.pad(W1r, ((0, KP - D_IN), (0, 0)))
    b1p = b1[None, :]
    w2l = jnp.pad(W2l, ((0, 0), (0, DP - D_OUT)))
    w2r = jnp.pad(W2r, ((0, 0), (0, DP - D_OUT)))
    b2p = jnp.pad(b2, (0, DP - D_OUT))[None, :]

    pad_ids = N + jnp.arange(EP - E, dtype=jnp.int32) % (NP - N)
    srcp = jnp.concatenate([edge_index[0], pad_ids])
    dstp = jnp.concatenate([edge_index[1], pad_ids])
    src2 = srcp.reshape(EP // CH, CH)
    dst2 = dstp.reshape(EP // CH, CH)
    src2d = srcp.reshape(EP // CHD, CHD)
    dst2d = dstp.reshape(EP // CHD, CHD)

    z128 = jnp.zeros((STRIPE, D_H), jnp.float32)

    y1, r1 = _tc_a(xp, w1l, w1r, b1p)
    cntp = _sc_cnt(dst2)
    aggp = _sc_b(y1, src2, dst2, z128)
    y2, r2 = _tc_c(aggp, cntp, r1, w2l, w2r, b2p)
    accp = _sc_d(y2, src2d, dst2d)
    o = _tc_e(accp, cntp, r2)
    return o[:N, :D_OUT]


# speedup vs baseline: 13.5938x; 1.0000x over previous
import functools

import jax
import jax.numpy as jnp
from jax import lax
from jax.experimental import pallas as pl
from jax.experimental.pallas import tpu as pltpu
from jax.experimental.pallas import tpu_sc as plsc

N = 10000
E = 320000
D_IN = 166
D_H = 128
D_OUT = 2

NP = 10240
KP = 176
DP = 16

NC = 2
NS = 16
NW = NC * NS

CH = 128
EP = 327680
EPW = EP // NW
NCHW = EPW // CH
SB1 = 2
SBCH = NCHW // SB1
STRIPE = NP // NS

BN = 1024
GRID = NP // BN


def _tc_a_body(x_ref, wl_ref, wr_ref, b_ref, y1_ref, r1_ref):
    x = x_ref[...]
    y1_ref[...] = jnp.dot(x, wl_ref[...], preferred_element_type=jnp.float32)
    r1_ref[...] = (
        jnp.dot(x, wr_ref[...], preferred_element_type=jnp.float32) + b_ref[...]
    )


def _tc_a(xp, wl, wr, b):
    return pl.pallas_call(
        _tc_a_body,
        grid=(GRID,),
        in_specs=[
            pl.BlockSpec((BN, KP), lambda i: (i, 0)),
            pl.BlockSpec((KP, D_H), lambda i: (0, 0)),
            pl.BlockSpec((KP, D_H), lambda i: (0, 0)),
            pl.BlockSpec((1, D_H), lambda i: (0, 0)),
        ],
        out_specs=[
            pl.BlockSpec((BN, D_H), lambda i: (i, 0)),
            pl.BlockSpec((BN, D_H), lambda i: (i, 0)),
        ],
        out_shape=[
            jax.ShapeDtypeStruct((NP, D_H), jnp.float32),
            jax.ShapeDtypeStruct((NP, D_H), jnp.float32),
        ],
    )(xp, wl, wr, b)


def _sc_b_body(y1_hbm, src_hbm, dst_hbm, z128_hbm,
               agg_out,
               srcm, dstm, rows0, rows1, agg_sh, sem0, sem1):
    c = lax.axis_index("c")
    s = lax.axis_index("s")
    wid = s * NC + c
    pltpu.sync_copy(z128_hbm, agg_sh.at[pl.ds(s * STRIPE, STRIPE)])
    plsc.subcore_barrier()

    for sb in range(SB1):
        rb = wid * NCHW + sb * SBCH
        pltpu.sync_copy(src_hbm.at[pl.ds(rb, SBCH)], srcm)
        pltpu.sync_copy(dst_hbm.at[pl.ds(rb, SBCH)], dstm)
        pltpu.async_copy(y1_hbm.at[srcm.at[0]], rows0, sem0)

        def body(k, carry):
            pltpu.async_copy(y1_hbm.at[srcm.at[2 * k + 1]], rows1, sem1)
            pltpu.make_async_copy(y1_hbm.at[srcm.at[2 * k]], rows0, sem0).wait()
            pltpu.sync_copy(rows0, agg_sh.at[dstm.at[2 * k]], add=True)

            @pl.when(k < SBCH // 2 - 1)
            def _():
                pltpu.async_copy(y1_hbm.at[srcm.at[2 * k + 2]], rows0, sem0)

            pltpu.make_async_copy(y1_hbm.at[srcm.at[2 * k + 1]], rows1,
                                  sem1).wait()
            pltpu.sync_copy(rows1, agg_sh.at[dstm.at[2 * k + 1]], add=True)
            return carry

        lax.fori_loop(0, SBCH // 2, body, 0)

    plsc.subcore_barrier()
    pltpu.sync_copy(agg_sh.at[pl.ds(s * STRIPE, STRIPE)],
                    agg_out.at[c, pl.ds(s * STRIPE, STRIPE)])


_sc_b = functools.partial(
    pl.kernel,
    mesh=plsc.VectorSubcoreMesh(core_axis_name="c", subcore_axis_name="s"),
    out_type=jax.ShapeDtypeStruct((NC, NP, D_H), jnp.float32),
    scratch_types=[
        pltpu.VMEM((SBCH, CH), jnp.int32),
        pltpu.VMEM((SBCH, CH), jnp.int32),
        pltpu.VMEM((CH, D_H), jnp.float32),
        pltpu.VMEM((CH, D_H), jnp.float32),
        pltpu.VMEM_SHARED((NP, D_H), jnp.float32),
        pltpu.SemaphoreType.DMA,
        pltpu.SemaphoreType.DMA,
    ],
)(_sc_b_body)


def _sc_cnt_body(dst_hbm,
                 cnt_out,
                 dstm, ones_v, cnt_sh, sem):
    c = lax.axis_index("c")
    s = lax.axis_index("s")
    wid = s * NC + c

    def fill0(i, carry):
        ones_v[i] = jnp.zeros((DP,), jnp.float32)
        return carry

    lax.fori_loop(0, CH, fill0, 0)
    for j in range(STRIPE // CH):
        pltpu.sync_copy(ones_v,
                        cnt_sh.at[pl.ds(s * STRIPE + j * CH, CH)])

    def fill1(i, carry):
        ones_v[i] = jnp.ones((DP,), jnp.float32)
        return carry

    lax.fori_loop(0, CH, fill1, 0)
    pltpu.sync_copy(dst_hbm.at[pl.ds(wid * NCHW, NCHW)], dstm)
    plsc.subcore_barrier()

    def body(g, carry):
        pltpu.async_copy(ones_v, cnt_sh.at[dstm.at[g]], sem, add=True)
        return carry

    lax.fori_loop(0, NCHW, body, 0)

    def drain(g, carry):
        pltpu.make_async_copy(ones_v, cnt_sh.at[dstm.at[0]], sem).wait()
        return carry

    lax.fori_loop(0, NCHW, drain, 0)
    plsc.subcore_barrier()
    pltpu.sync_copy(cnt_sh.at[pl.ds(s * STRIPE, STRIPE)],
                    cnt_out.at[c, pl.ds(s * STRIPE, STRIPE)])


_sc_cnt = functools.partial(
    pl.kernel,
    mesh=plsc.VectorSubcoreMesh(core_axis_name="c", subcore_axis_name="s"),
    out_type=jax.ShapeDtypeStruct((NC, NP, DP), jnp.float32),
    compiler_params=pltpu.CompilerParams(use_tc_tiling_on_sc=False),
    scratch_types=[
        pltpu.VMEM((NCHW, CH), jnp.int32),
        pltpu.VMEM((CH, DP), jnp.float32),
        pltpu.VMEM_SHARED((NP, DP), jnp.float32),
        pltpu.SemaphoreType.DMA,
    ],
)(_sc_cnt_body)


def _tc_c_body(agg_ref, cnt_ref, r1_ref, wl_ref, wr_ref, b_ref,
               y2_ref, r2_ref):
    cnt = cnt_ref[0, :, 0:1] + cnt_ref[1, :, 0:1]
    mean = (agg_ref[0] + agg_ref[1]) / jnp.maximum(cnt, 1.0)
    h = jnp.maximum(mean + r1_ref[...], 0.0)
    y2_ref[...] = jnp.dot(h, wl_ref[...], preferred_element_type=jnp.float32)
    r2_ref[...] = (
        jnp.dot(h, wr_ref[...], preferred_element_type=jnp.float32) + b_ref[...]
    )


def _tc_c(aggp, cntp, r1, wl, wr, b):
    return pl.pallas_call(
        _tc_c_body,
        grid=(GRID,),
        in_specs=[
            pl.BlockSpec((NC, BN, D_H), lambda i: (0, i, 0)),
            pl.BlockSpec((NC, BN, DP), lambda i: (0, i, 0)),
            pl.BlockSpec((BN, D_H), lambda i: (i, 0)),
            pl.BlockSpec((D_H, DP), lambda i: (0, 0)),
            pl.BlockSpec((D_H, DP), lambda i: (0, 0)),
            pl.BlockSpec((1, DP), lambda i: (0, 0)),
        ],
        out_specs=[
            pl.BlockSpec((BN, DP), lambda i: (i, 0)),
            pl.BlockSpec((BN, DP), lambda i: (i, 0)),
        ],
        out_shape=[
            jax.ShapeDtypeStruct((NP, DP), jnp.float32),
            jax.ShapeDtypeStruct((NP, DP), jnp.float32),
        ],
    )(aggp, cntp, r1, wl, wr, b)


CHD = 512
NCHD = EPW // CHD


def _sc_d_body(y2_hbm, src_hbm, dst_hbm,
               acc_out,
               srcm, dstm, rows0, rows1, acc_sh, sem0, sem1):
    c = lax.axis_index("c")
    s = lax.axis_index("s")
    wid = s * NC + c

    def fill0(i, carry):
        rows0[i] = jnp.zeros((DP,), jnp.float32)
        return carry

    lax.fori_loop(0, CHD, fill0, 0)
    for j in range(STRIPE // CHD):
        pltpu.sync_copy(rows0.at[pl.ds(0, CHD)],
                        acc_sh.at[pl.ds(s * STRIPE + j * CHD, CHD)])
    pltpu.sync_copy(rows0.at[pl.ds(0, STRIPE - (STRIPE // CHD) * CHD)],
                    acc_sh.at[pl.ds(s * STRIPE + (STRIPE // CHD) * CHD,
                                    STRIPE - (STRIPE // CHD) * CHD)])
    plsc.subcore_barrier()

    rb = wid * NCHD
    pltpu.sync_copy(src_hbm.at[pl.ds(rb, NCHD)], srcm)
    pltpu.sync_copy(dst_hbm.at[pl.ds(rb, NCHD)], dstm)
    pltpu.async_copy(y2_hbm.at[srcm.at[0]], rows0, sem0)

    def body(k, carry):
        pltpu.async_copy(y2_hbm.at[srcm.at[2 * k + 1]], rows1, sem1)
        pltpu.make_async_copy(y2_hbm.at[srcm.at[2 * k]], rows0,
                              sem0).wait()
        pltpu.sync_copy(rows0, acc_sh.at[dstm.at[2 * k]], add=True)

        @pl.when(k < NCHD // 2 - 1)
        def _():
            pltpu.async_copy(y2_hbm.at[srcm.at[2 * k + 2]], rows0, sem0)

        pltpu.make_async_copy(y2_hbm.at[srcm.at[2 * k + 1]], rows1,
                              sem1).wait()
        pltpu.sync_copy(rows1, acc_sh.at[dstm.at[2 * k + 1]], add=True)
        return carry

    lax.fori_loop(0, NCHD // 2, body, 0)
    plsc.subcore_barrier()
    pltpu.sync_copy(acc_sh.at[pl.ds(s * STRIPE, STRIPE)],
                    acc_out.at[c, pl.ds(s * STRIPE, STRIPE)])


_sc_d = functools.partial(
    pl.kernel,
    mesh=plsc.VectorSubcoreMesh(core_axis_name="c", subcore_axis_name="s"),
    out_type=jax.ShapeDtypeStruct((NC, NP, DP), jnp.float32),
    compiler_params=pltpu.CompilerParams(use_tc_tiling_on_sc=False),
    scratch_types=[
        pltpu.VMEM((NCHD, CHD), jnp.int32),
        pltpu.VMEM((NCHD, CHD), jnp.int32),
        pltpu.VMEM((CHD, DP), jnp.float32),
        pltpu.VMEM((CHD, DP), jnp.float32),
        pltpu.VMEM_SHARED((NP, DP), jnp.float32),
        pltpu.SemaphoreType.DMA,
        pltpu.SemaphoreType.DMA,
    ],
)(_sc_d_body)


def _tc_e_body(acc_ref, cnt_ref, r2_ref, o_ref):
    cnt = cnt_ref[0, :, 0:1] + cnt_ref[1, :, 0:1]
    z = (acc_ref[0] + acc_ref[1]) / jnp.maximum(cnt, 1.0) + r2_ref[...]
    z0 = z[:, 0:1]
    z1 = z[:, 1:2]
    m = jnp.maximum(z0, z1)
    lse = m + jnp.log(jnp.exp(z0 - m) + jnp.exp(z1 - m))
    o_ref[...] = z - lse


def _tc_e(accp, cntp, r2):
    return pl.pallas_call(
        _tc_e_body,
        grid=(GRID,),
        in_specs=[
            pl.BlockSpec((NC, BN, DP), lambda i: (0, i, 0)),
            pl.BlockSpec((NC, BN, DP), lambda i: (0, i, 0)),
            pl.BlockSpec((BN, DP), lambda i: (i, 0)),
        ],
        out_specs=pl.BlockSpec((BN, DP), lambda i: (i, 0)),
        out_shape=jax.ShapeDtypeStruct((NP, DP), jnp.float32),
    )(accp, cntp, r2)


def kernel(x, edge_index, W1l, b1, W1r, W2l, b2, W2r):
    xp = jnp.pad(x, ((0, NP - N), (0, KP - D_IN)))
    w1l = jnp.pad(W1l, ((0, KP - D_IN), (0, 0)))
    w1r = jnp.pad(W1r, ((0, KP - D_IN), (0, 0)))
    b1p = b1[None, :]
    w2l = jnp.pad(W2l, ((0, 0), (0, DP - D_OUT)))
    w2r = jnp.pad(W2r, ((0, 0), (0, DP - D_OUT)))
    b2p = jnp.pad(b2, (0, DP - D_OUT))[None, :]

    pad_ids = N + jnp.arange(EP - E, dtype=jnp.int32) % (NP - N)
    srcp = jnp.concatenate([edge_index[0], pad_ids])
    dstp = jnp.concatenate([edge_index[1], pad_ids])
    src2 = srcp.reshape(EP // CH, CH)
    dst2 = dstp.reshape(EP // CH, CH)
    src2d = srcp.reshape(EP // CHD, CHD)
    dst2d = dstp.reshape(EP // CHD, CHD)

    z128 = jnp.zeros((STRIPE, D_H), jnp.float32)

    y1, r1 = _tc_a(xp, w1l, w1r, b1p)
    cntp = _sc_cnt(dst2)
    aggp = _sc_b(y1, src2, dst2, z128)
    y2, r2 = _tc_c(aggp, cntp, r1, w2l, w2r, b2p)
    accp = _sc_d(y2, src2d, dst2d)
    o = _tc_e(accp, cntp, r2)
    return o[:N, :D_OUT]
